# Initial kernel scaffold; baseline (speedup 1.0000x reference)
import jax
import jax.numpy as jnp
from jax.experimental import pallas as pl


def kernel(x, w_padded, b_padded):
    raise NotImplementedError("write your pallas kernel here")


# speedup vs baseline: 1.0310x; 1.0310x over previous
import functools

import jax
import jax.numpy as jnp
from jax.experimental import pallas as pl
from jax.experimental.pallas import tpu as pltpu


def _round_up(x, m):
    return ((x + m - 1) // m) * m


def _conv_stack_kernel(x_ref, w_ref, b_ref, o_ref, act_a, act_b, *,
                       n_layers, ksize, seq_len, pad_lo, front):
    bt, _, cp = x_ref.shape
    m = bt * seq_len
    base = front - pad_lo
    halo_len = act_a.shape[1]
    tail = front + seq_len

    for buf in (act_a, act_b):
        buf[:, pl.ds(0, front), :] = jnp.zeros((bt, front, cp), buf.dtype)
        if halo_len > tail:
            buf[:, pl.ds(tail, halo_len - tail), :] = jnp.zeros(
                (bt, halo_len - tail, cp), buf.dtype)
    act_a[:, pl.ds(front, seq_len), :] = x_ref[...]

    bufs = (act_a, act_b)
    for layer in range(n_layers):
        src = bufs[layer % 2]
        dst = bufs[(layer + 1) % 2]

        acc = None
        for k in range(ksize):
            lhs = src[:, pl.ds(base + k, seq_len), :].reshape(m, cp)
            d = jnp.dot(lhs, w_ref[layer, k],
                        preferred_element_type=jnp.float32)
            acc = d if acc is None else acc + d
        y = acc + b_ref[layer].astype(jnp.float32)

        if layer == n_layers - 1:
            o_ref[...] = y.reshape(bt, seq_len, cp).astype(o_ref.dtype)
        else:
            dst[:, pl.ds(front, seq_len), :] = (
                y.reshape(bt, seq_len, cp).astype(dst.dtype))


def _conv_stack(x_blc, w, b):
    B, L, Cp = x_blc.shape
    n_layers, K, _, _ = w.shape
    pad_lo = (K - 1) // 2
    pad_hi = K - 1 - pad_lo
    front = _round_up(max(pad_lo, 1), 8)
    bt = min(B, max(1, 256 // max(L, 1)))
    Bp = _round_up(B, bt)
    if Bp != B:
        x_blc = jnp.pad(x_blc, ((0, Bp - B), (0, 0), (0, 0)))
    grid = (Bp // bt,)
    halo_len = front + L + pad_hi

    fn = functools.partial(
        _conv_stack_kernel, n_layers=n_layers, ksize=K,
        seq_len=L, pad_lo=pad_lo, front=front)
    out = pl.pallas_call(
        fn,
        out_shape=jax.ShapeDtypeStruct((Bp, L, Cp), x_blc.dtype),
        grid_spec=pltpu.PrefetchScalarGridSpec(
            num_scalar_prefetch=0,
            grid=grid,
            in_specs=[
                pl.BlockSpec((bt, L, Cp), lambda i: (i, 0, 0)),
                pl.BlockSpec((n_layers, K, Cp, Cp), lambda i: (0, 0, 0, 0)),
                pl.BlockSpec((n_layers, 1, Cp), lambda i: (0, 0, 0)),
            ],
            out_specs=pl.BlockSpec((bt, L, Cp), lambda i: (i, 0, 0)),
            scratch_shapes=[
                pltpu.VMEM((bt, halo_len, Cp), x_blc.dtype),
                pltpu.VMEM((bt, halo_len, Cp), x_blc.dtype),
            ],
        ),
        compiler_params=pltpu.CompilerParams(
            dimension_semantics=("parallel",),
            vmem_limit_bytes=48 * 1024 * 1024,
        ),
    )(x_blc, w, b)
    return out[:B]


def kernel(x, w_padded, b_padded):
    B, C, L = x.shape
    Cp = w_padded.shape[-1]
    xt = jnp.transpose(x, (0, 2, 1))
    if Cp != C:
        xt = jnp.pad(xt, ((0, 0), (0, 0), (0, Cp - C)))
    y = _conv_stack(xt, w_padded, b_padded)
    return jnp.transpose(y[:, :, :C], (0, 2, 1))


# speedup vs baseline: 1.5213x; 1.4756x over previous
import functools

import jax
import jax.numpy as jnp
from jax.experimental import pallas as pl
from jax.experimental.pallas import tpu as pltpu


def _round_up(x, m):
    return ((x + m - 1) // m) * m


def _conv_stack_kernel(x_ref, w_ref, b_ref, bcol_ref, eye_ref, o_ref,
                       act_a, act_b, *, n_layers, ksize, seq_len, pad_lo,
                       front):
    bt = x_ref.shape[0]
    cp = x_ref.shape[1]
    m = bt * seq_len
    base = front - pad_lo
    halo_len = act_a.shape[1]
    tail = front + seq_len

    for buf in (act_a, act_b):
        buf[:, pl.ds(0, front), :] = jnp.zeros((bt, front, cp), buf.dtype)
        if halo_len > tail:
            buf[:, pl.ds(tail, halo_len - tail), :] = jnp.zeros(
                (bt, halo_len - tail, cp), buf.dtype)

    xcat = jnp.concatenate([x_ref[i] for i in range(bt)], axis=1)
    s0 = jax.lax.dot_general(
        xcat, eye_ref[...], (((0,), (0,)), ((), ())),
        preferred_element_type=jnp.float32)
    act_a[:, pl.ds(front, seq_len), :] = s0.reshape(bt, seq_len, cp)

    bufs = (act_a, act_b)
    for layer in range(n_layers):
        src = bufs[layer % 2]
        dst = bufs[(layer + 1) % 2]
        last = layer == n_layers - 1

        acc = None
        for k in range(ksize):
            lhs = src[:, pl.ds(base + k, seq_len), :].reshape(m, cp)
            if last:
                d = jax.lax.dot_general(
                    w_ref[layer, k], lhs, (((0,), (1,)), ((), ())),
                    preferred_element_type=jnp.float32)
            else:
                d = jnp.dot(lhs, w_ref[layer, k],
                            preferred_element_type=jnp.float32)
            acc = d if acc is None else acc + d

        if last:
            y = acc + bcol_ref[layer].astype(jnp.float32)
            for i in range(bt):
                o_ref[i] = y[:, i * seq_len:(i + 1) * seq_len]
        else:
            y = acc + b_ref[layer].astype(jnp.float32)
            dst[:, pl.ds(front, seq_len), :] = y.reshape(bt, seq_len, cp)


def kernel(x, w_padded, b_padded):
    B, C, L = x.shape
    n_layers, K, _, Cp = w_padded.shape
    pad_lo = (K - 1) // 2
    pad_hi = K - 1 - pad_lo
    front = _round_up(max(pad_lo, 1), 8)
    bt = min(B, max(1, 512 // max(L, 1)))
    Bp = _round_up(B, bt)
    if Bp != B:
        x = jnp.pad(x, ((0, Bp - B), (0, 0), (0, 0)))
    grid = (Bp // bt,)
    halo_len = front + L + pad_hi

    b_col = jnp.swapaxes(b_padded, 1, 2)
    eye = jnp.eye(Cp, dtype=x.dtype)

    fn = functools.partial(
        _conv_stack_kernel, n_layers=n_layers, ksize=K,
        seq_len=L, pad_lo=pad_lo, front=front)
    out = pl.pallas_call(
        fn,
        out_shape=jax.ShapeDtypeStruct((Bp, Cp, L), x.dtype),
        grid_spec=pltpu.PrefetchScalarGridSpec(
            num_scalar_prefetch=0,
            grid=grid,
            in_specs=[
                pl.BlockSpec((bt, Cp, L), lambda i: (i, 0, 0)),
                pl.BlockSpec((n_layers, K, Cp, Cp), lambda i: (0, 0, 0, 0)),
                pl.BlockSpec((n_layers, 1, Cp), lambda i: (0, 0, 0)),
                pl.BlockSpec((n_layers, Cp, 1), lambda i: (0, 0, 0)),
                pl.BlockSpec((Cp, Cp), lambda i: (0, 0)),
            ],
            out_specs=pl.BlockSpec((bt, Cp, L), lambda i: (i, 0, 0)),
            scratch_shapes=[
                pltpu.VMEM((bt, halo_len, Cp), x.dtype),
                pltpu.VMEM((bt, halo_len, Cp), x.dtype),
            ],
        ),
        compiler_params=pltpu.CompilerParams(
            dimension_semantics=("parallel",),
            vmem_limit_bytes=48 * 1024 * 1024,
        ),
    )(x, w_padded, b_padded, b_col, eye)
    return out[:B]


# speedup vs baseline: 1.5767x; 1.0364x over previous
import functools

import jax
import jax.numpy as jnp
from jax.experimental import pallas as pl
from jax.experimental.pallas import tpu as pltpu


def _round_up(x, m):
    return ((x + m - 1) // m) * m


def _conv_stack_kernel(x_ref, w_ref, b_ref, bcol_ref, eye_ref, o_ref,
                       act_a, act_b, *, n_layers, ksize, seq_len,
                       pad_lo, front):
    bt = x_ref.shape[0]
    cp = x_ref.shape[1]
    m = bt * seq_len
    base = front - pad_lo
    halo_len = act_a.shape[1]
    tail = front + seq_len

    for buf in (act_a, act_b):
        buf[:, pl.ds(0, front), :] = jnp.zeros((bt, front, cp), buf.dtype)
        if halo_len > tail:
            buf[:, pl.ds(tail, halo_len - tail), :] = jnp.zeros(
                (bt, halo_len - tail, cp), buf.dtype)

    xcat = jnp.concatenate([x_ref[i] for i in range(bt)], axis=1)
    s0 = jax.lax.dot_general(
        xcat, eye_ref[...], (((0,), (0,)), ((), ())),
        preferred_element_type=jnp.float32)
    act_a[:, pl.ds(front, seq_len), :] = (
        s0.reshape(bt, seq_len, cp).astype(act_a.dtype))

    bufs = (act_a, act_b)
    for layer in range(n_layers):
        src = bufs[layer % 2]
        dst = bufs[(layer + 1) % 2]
        last = layer == n_layers - 1

        acc = None
        for k in range(ksize):
            lhs = src[:, pl.ds(base + k, seq_len), :].reshape(m, cp)
            if last:
                d = jax.lax.dot_general(
                    w_ref[layer, k], lhs, (((0,), (1,)), ((), ())),
                    preferred_element_type=jnp.float32)
            else:
                d = jnp.dot(lhs, w_ref[layer, k],
                            preferred_element_type=jnp.float32)
            acc = d if acc is None else acc + d

        if last:
            y = acc + bcol_ref[layer].astype(jnp.float32)
            for i in range(bt):
                o_ref[i] = y[:, i * seq_len:(i + 1) * seq_len]
        else:
            y = acc + b_ref[layer].astype(jnp.float32)
            dst[:, pl.ds(front, seq_len), :] = (
                y.reshape(bt, seq_len, cp).astype(dst.dtype))


def kernel(x, w_padded, b_padded):
    B, C, L = x.shape
    n_layers, K, _, Cp = w_padded.shape
    pad_lo = (K - 1) // 2
    pad_hi = K - 1 - pad_lo
    front = _round_up(max(pad_lo, 1), 8)
    bt = min(B, max(1, 1024 // max(L, 1)))
    Bp = _round_up(B, bt)
    if Bp != B:
        x = jnp.pad(x, ((0, Bp - B), (0, 0), (0, 0)))
    grid = (Bp // bt,)
    halo_len = front + L + pad_hi

    b_col = jnp.swapaxes(b_padded, 1, 2)
    eye = jnp.eye(Cp, dtype=x.dtype)

    fn = functools.partial(
        _conv_stack_kernel, n_layers=n_layers, ksize=K,
        seq_len=L, pad_lo=pad_lo, front=front)
    out = pl.pallas_call(
        fn,
        out_shape=jax.ShapeDtypeStruct((Bp, Cp, L), x.dtype),
        grid_spec=pltpu.PrefetchScalarGridSpec(
            num_scalar_prefetch=0,
            grid=grid,
            in_specs=[
                pl.BlockSpec((bt, Cp, L), lambda i: (i, 0, 0)),
                pl.BlockSpec((n_layers, K, Cp, Cp), lambda i: (0, 0, 0, 0)),
                pl.BlockSpec((n_layers, 1, Cp), lambda i: (0, 0, 0)),
                pl.BlockSpec((n_layers, Cp, 1), lambda i: (0, 0, 0)),
                pl.BlockSpec((Cp, Cp), lambda i: (0, 0)),
            ],
            out_specs=pl.BlockSpec((bt, Cp, L), lambda i: (i, 0, 0)),
            scratch_shapes=[
                pltpu.VMEM((bt, halo_len, Cp), x.dtype),
                pltpu.VMEM((bt, halo_len, Cp), x.dtype),
            ],
        ),
        compiler_params=pltpu.CompilerParams(
            dimension_semantics=("parallel",),
            vmem_limit_bytes=56 * 1024 * 1024,
        ),
    )(x, w_padded, b_padded, b_col, eye)
    return out[:B]
